# Initial kernel scaffold; baseline (speedup 1.0000x reference)
import jax
import jax.numpy as jnp
from jax.experimental import pallas as pl


def kernel(base_feat, im_info, gt_boxes, W1, b1, Wc, bc, Wb, bb):
    raise NotImplementedError("write your pallas kernel here")


# speedup vs baseline: 12.5330x; 12.5330x over previous
import numpy as np
import jax
import jax.numpy as jnp
from jax.experimental import pallas as pl
from jax.experimental.pallas import tpu as pltpu

_STRIDE = 16
_PRE_NMS = 6000
_POST_NMS = 300
_NMS_THRESH = 0.7
_MIN_SIZE = 16.0
_NEG_INF = float("-inf")
_INT_MIN = np.int32(-2147483648)
_NR = 288
_NL = 128


def _gen_base_anchors():
    ratios = np.array([0.5, 1.0, 2.0])
    scales = np.array([8.0, 16.0, 32.0])
    base_anchor = np.array([0, 0, 15, 15], dtype=np.float64)
    w = base_anchor[2] - base_anchor[0] + 1
    h = base_anchor[3] - base_anchor[1] + 1
    x_ctr = base_anchor[0] + 0.5 * (w - 1)
    y_ctr = base_anchor[1] + 0.5 * (h - 1)
    size = w * h
    ws = np.round(np.sqrt(size / ratios))
    hs = np.round(ws * ratios)
    out = []
    for i in range(3):
        ax1 = x_ctr - 0.5 * (ws[i] - 1)
        ay1 = y_ctr - 0.5 * (hs[i] - 1)
        ax2 = x_ctr + 0.5 * (ws[i] - 1)
        ay2 = y_ctr + 0.5 * (hs[i] - 1)
        aw = ax2 - ax1 + 1
        ah = ay2 - ay1 + 1
        acx = ax1 + 0.5 * (aw - 1)
        acy = ay1 + 0.5 * (ah - 1)
        sw = aw * scales
        sh = ah * scales
        for s in range(3):
            out.append([acx - 0.5 * (sw[s] - 1), acy - 0.5 * (sh[s] - 1),
                        acx + 0.5 * (sw[s] - 1), acy + 0.5 * (sh[s] - 1)])
    return np.array(out, dtype=np.float32)


_BASE_ANCHORS = _gen_base_anchors()


def _anchor_tables():
    pp = np.arange(36864)
    hw = pp // 9
    a = pp % 9
    j = (hw % 64).astype(np.float32) * _STRIDE
    i = (hw // 64).astype(np.float32) * _STRIDE
    b = _BASE_ANCHORS[a]
    w = b[:, 2] - b[:, 0] + 1.0
    h = b[:, 3] - b[:, 1] + 1.0
    cx = (b[:, 0] + j) + 0.5 * w
    cy = (b[:, 1] + i) + 0.5 * h
    return np.stack([w, h, cx, cy]).reshape(4, _NR, _NL).astype(np.float32)


_ANCH = _anchor_tables()


def _prop_kernel(fg_ref, d_ref, info_ref, anch_ref, out_ref):
    f32 = jnp.float32
    fg = fg_ref[:]
    dxv = d_ref[:, 0]
    dyv = d_ref[:, 1]
    dwv = d_ref[:, 2]
    dhv = d_ref[:, 3]

    p = (jax.lax.broadcasted_iota(jnp.int32, (1, _NR, _NL), 1) * _NL +
         jax.lax.broadcasted_iota(jnp.int32, (1, _NR, _NL), 2))
    wa = anch_ref[0][None]
    ha = anch_ref[1][None]
    cx = anch_ref[2][None]
    cy = anch_ref[3][None]

    imh = info_ref[:, 0:1][:, :, None]
    imw = info_ref[:, 1:2][:, :, None]
    scale = info_ref[:, 2:3][:, :, None]

    pcx = dxv * wa + cx
    pcy = dyv * ha + cy
    pw = jnp.exp(dwv) * wa
    ph = jnp.exp(dhv) * ha
    x1 = jnp.clip(pcx - 0.5 * pw, 0.0, imw - 1.0)
    y1 = jnp.clip(pcy - 0.5 * ph, 0.0, imh - 1.0)
    x2 = jnp.clip(pcx + 0.5 * pw, 0.0, imw - 1.0)
    y2 = jnp.clip(pcy + 0.5 * ph, 0.0, imh - 1.0)

    ms = _MIN_SIZE * scale
    valid = ((x2 - x1 + 1.0) >= ms) & ((y2 - y1 + 1.0) >= ms)
    sm = jnp.where(valid, fg, _NEG_INF)

    bits = jax.lax.bitcast_convert_type(sm, jnp.int32)
    key = bits ^ (jax.lax.shift_right_arithmetic(bits, 31) &
                  np.int32(0x7FFFFFFF))

    def _cnt(th):
        c = (key >= th).astype(jnp.int32)
        return jnp.sum(jnp.sum(c, axis=2, keepdims=True), axis=1,
                       keepdims=True)

    cnt0 = _cnt(jnp.zeros((2, 1, 1), jnp.int32))
    theta = jnp.where(cnt0 >= _PRE_NMS, np.int32(0), _INT_MIN)
    for bit in range(30, -1, -1):
        t = theta | np.int32(1 << bit)
        theta = jnp.where(_cnt(t) >= _PRE_NMS, t, theta)

    s0 = jnp.where(key >= theta, sm, _NEG_INF)

    areas = (x2 - x1 + 1.0) * (y2 - y1 + 1.0)
    li = jax.lax.broadcasted_iota(jnp.int32, (2, 128), 1)
    bidx = jax.lax.broadcasted_iota(jnp.int32, (2, 128), 0).astype(f32)

    def body(i, carry):
        s, fb_row = carry
        m = jnp.max(jnp.max(s, axis=2, keepdims=True), axis=1, keepdims=True)
        ismax = s == m
        bi = jnp.min(jnp.min(jnp.where(ismax, p, np.int32(1 << 30)),
                             axis=2, keepdims=True), axis=1, keepdims=True)
        bm = p == bi

        def pick(arr):
            return jnp.sum(jnp.sum(jnp.where(bm, arr, 0.0), axis=2,
                                   keepdims=True), axis=1, keepdims=True)

        cx1 = pick(x1)
        cy1 = pick(y1)
        cx2 = pick(x2)
        cy2 = pick(y2)

        xx1 = jnp.maximum(cx1, x1)
        yy1 = jnp.maximum(cy1, y1)
        xx2 = jnp.minimum(cx2, x2)
        yy2 = jnp.minimum(cy2, y2)
        iw = jnp.maximum(0.0, xx2 - xx1 + 1.0)
        ih = jnp.maximum(0.0, yy2 - yy1 + 1.0)
        inter = iw * ih
        barea = (cx2 - cx1 + 1.0) * (cy2 - cy1 + 1.0)
        iou = inter / (barea + areas - inter)
        s2 = jnp.where(iou > _NMS_THRESH, _NEG_INF, s)
        s2 = jnp.where(bm, _NEG_INF, s2)

        cur = (jnp.where(li == 0, bidx, 0.0) +
               jnp.where(li == 1, cx1[:, :, 0], 0.0) +
               jnp.where(li == 2, cy1[:, :, 0], 0.0) +
               jnp.where(li == 3, cx2[:, :, 0], 0.0) +
               jnp.where(li == 4, cy2[:, :, 0], 0.0))
        fb2 = jnp.where(i == 0, cur, fb_row)
        neg = (m == _NEG_INF)[:, :, 0]
        rowout = jnp.where(neg, fb2, cur)
        out_ref[:, pl.ds(i, 1), :] = rowout[:, None, :]
        return (s2, fb2)

    jax.lax.fori_loop(0, _POST_NMS, body,
                      (s0, jnp.zeros((2, 128), jnp.float32)))


def _conv2d(x, w, b, pad):
    y = jax.lax.conv_general_dilated(
        x, w, (1, 1), [(pad, pad), (pad, pad)],
        dimension_numbers=('NCHW', 'OIHW', 'NCHW'))
    return y + b[None, :, None, None]


def kernel(base_feat, im_info, gt_boxes, W1, b1, Wc, bc, Wb, bb):
    del gt_boxes
    B = base_feat.shape[0]
    x = jax.nn.relu(_conv2d(base_feat, W1, b1, 1))
    scores = _conv2d(x, Wc, bc, 0)
    bbox_pred = _conv2d(x, Wb, bb, 0)
    fg = jnp.transpose(scores[:, 9:, :, :], (0, 2, 3, 1)).reshape(B, -1)
    deltas = jnp.transpose(bbox_pred, (0, 2, 3, 1)).reshape(B, -1, 4)

    fg_r = fg.reshape(B, _NR, _NL)
    d_r = deltas.transpose(0, 2, 1).reshape(B, 4, _NR, _NL)
    info_pad = jnp.zeros((B, 128), jnp.float32).at[:, :3].set(im_info)
    anch = jnp.asarray(_ANCH)

    rois_pad = pl.pallas_call(
        _prop_kernel,
        out_shape=jax.ShapeDtypeStruct((B, 304, 128), jnp.float32),
        compiler_params=pltpu.CompilerParams(
            vmem_limit_bytes=100 * 1024 * 1024),
    )(fg_r, d_r, info_pad, anch)
    return rois_pad[:, :300, :5]
